# baseline (speedup 1.0000x reference)
import functools

import jax
import jax.numpy as jnp
from jax import lax
from jax.experimental import pallas as pl
from jax.experimental.pallas import tpu as pltpu
from jax.experimental.pallas import tpu_sc as plsc

_B = 1024
_M = 64
_D = 256

_NC = 2
_NS = 16
_NW = _NC * _NS

_ROWS_PER_W = (_B * _M) // _NW
_CH = 128
_NCH = _ROWS_PER_W // _CH
_ZB = _B // _NW

_BB = 32


def _sc_gather_body(wr_hbm, rid_hbm, zq_hbm, qrid_hbm, g_out, zq_out,
                    idx_v, rows_a, rows_b, zidx_v, zrows_v, sem_a, sem_b):
    wid = lax.axis_index("s") * _NC + lax.axis_index("c")
    base = wid * _ROWS_PER_W
    pltpu.sync_copy(rid_hbm.at[pl.ds(base, _ROWS_PER_W)], idx_v)
    bufs = (rows_a, rows_b)
    sems = (sem_a, sem_b)
    cps = []
    for c in range(_NCH):
        cps.append(pltpu.async_copy(
            wr_hbm.at[idx_v.at[pl.ds(c * _CH, _CH)]], bufs[c % 2], sems[c % 2]))
        if c > 0:
            cps[c - 1].wait()
            pltpu.sync_copy(bufs[(c - 1) % 2],
                            g_out.at[pl.ds(base + (c - 1) * _CH, _CH)])
    cps[_NCH - 1].wait()
    pltpu.sync_copy(bufs[(_NCH - 1) % 2],
                    g_out.at[pl.ds(base + (_NCH - 1) * _CH, _CH)])
    zbase = wid * _ZB
    pltpu.sync_copy(qrid_hbm.at[pl.ds(zbase, _ZB)], zidx_v)
    pltpu.async_copy(zq_hbm.at[zidx_v], zrows_v, sem_a).wait()
    pltpu.sync_copy(zrows_v, zq_out.at[pl.ds(zbase, _ZB)])


@jax.jit
def _sc_gather(w_r_table, rid_flat, zq_table, qrid):
    mesh = plsc.VectorSubcoreMesh(core_axis_name="c", subcore_axis_name="s")
    return pl.kernel(
        _sc_gather_body,
        mesh=mesh,
        out_type=[
            jax.ShapeDtypeStruct((_B * _M, _D), jnp.float32),
            jax.ShapeDtypeStruct((_B, _D), jnp.float32),
        ],
        scratch_types=[
            pltpu.VMEM((_ROWS_PER_W,), jnp.int32),
            pltpu.VMEM((_CH, _D), jnp.float32),
            pltpu.VMEM((_CH, _D), jnp.float32),
            pltpu.VMEM((_ZB,), jnp.int32),
            pltpu.VMEM((_ZB, _D), jnp.float32),
            pltpu.SemaphoreType.DMA,
            pltpu.SemaphoreType.DMA,
        ],
    )(w_r_table, rid_flat, zq_table, qrid)


def _tc_body(e_ref, g_ref, zq_ref, rw_ref, wattn_ref, battn_ref, wua_ref,
             out_ref):
    bb, m, d = e_ref.shape
    e = e_ref[...].reshape(bb * m, d)
    g = g_ref[...].reshape(bb * m, d)
    nrm = jnp.sqrt(jnp.sum(g * g, axis=1, keepdims=True))
    gn = g / jnp.maximum(nrm, 1e-12)
    dcoef = jnp.sum(e * gn, axis=1, keepdims=True)
    etr = e - dcoef * gn
    w1 = wattn_ref[:, :d].astype(jnp.bfloat16)
    w2 = wattn_ref[:, d:].astype(jnp.bfloat16)
    zqw = lax.dot_general(zq_ref[...].astype(jnp.bfloat16), w1,
                          (((1,), (1,)), ((), ())),
                          preferred_element_type=jnp.float32)
    wa = lax.dot_general(etr.astype(jnp.bfloat16), w2,
                         (((1,), (1,)), ((), ())),
                         preferred_element_type=jnp.float32)
    wa = wa.reshape(bb, m, 2 * d) + zqw[:, None, :] + battn_ref[...][None]
    t = jnp.tanh(wa)
    logits = jnp.sum(t * wua_ref[...][None], axis=2)
    logits = logits - jnp.max(logits, axis=1, keepdims=True)
    ex = jnp.exp(logits)
    alpha = ex / jnp.sum(ex, axis=1, keepdims=True)
    attn = alpha + rw_ref[...]
    out_ref[...] = jnp.sum(attn[:, :, None] * etr.reshape(bb, m, d), axis=1)


@jax.jit
def _tc_attention(e, g, zq_rows, rw, W_attn, b_attn2d, W_ua):
    grid = (_B // _BB,)
    return pl.pallas_call(
        _tc_body,
        grid=grid,
        in_specs=[
            pl.BlockSpec((_BB, _M, _D), lambda i: (i, 0, 0)),
            pl.BlockSpec((_BB, _M, _D), lambda i: (i, 0, 0)),
            pl.BlockSpec((_BB, _D), lambda i: (i, 0)),
            pl.BlockSpec((_BB, _M), lambda i: (i, 0)),
            pl.BlockSpec((2 * _D, 2 * _D), lambda i: (0, 0)),
            pl.BlockSpec((1, 2 * _D), lambda i: (0, 0)),
            pl.BlockSpec((1, 2 * _D), lambda i: (0, 0)),
        ],
        out_specs=pl.BlockSpec((_BB, _D), lambda i: (i, 0)),
        out_shape=jax.ShapeDtypeStruct((_B, _D), jnp.float32),
    )(e, g, zq_rows, rw, W_attn, b_attn2d, W_ua)


def kernel(batch_nei_rid, batch_nei_e_emb, batch_nei_rw, batch_q_rid,
           w_r_table, zq_table, W_attn, b_attn, W_ua, b_ua):
    del b_ua
    rid_flat = batch_nei_rid.reshape(_B * _M).astype(jnp.int32)
    qrid = batch_q_rid.astype(jnp.int32)
    g_flat, zq_rows = _sc_gather(w_r_table, rid_flat, zq_table, qrid)
    g = g_flat.reshape(_B, _M, _D)
    return _tc_attention(batch_nei_e_emb, g, zq_rows, batch_nei_rw,
                         W_attn, b_attn.reshape(1, 2 * _D), W_ua)


# speedup vs baseline: 1.1653x; 1.1653x over previous
import functools

import jax
import jax.numpy as jnp
from jax import lax
from jax.experimental import pallas as pl
from jax.experimental.pallas import tpu as pltpu
from jax.experimental.pallas import tpu_sc as plsc

_B = 1024
_M = 64
_D = 256

_NC = 2
_NS = 16
_NW = _NC * _NS

_ROWS_PER_W = (_B * _M) // _NW
_CH = 128
_NCH = _ROWS_PER_W // _CH
_ZB = _B // _NW

_BB = 64


def _sc_gather_body(wr_hbm, rid_hbm, zq_hbm, qrid_hbm, g_out, zq_out,
                    idx_v, rows_a, rows_b, zidx_v, zrows_v, sem_a, sem_b):
    wid = lax.axis_index("s") * _NC + lax.axis_index("c")
    base = wid * _ROWS_PER_W
    pltpu.sync_copy(rid_hbm.at[pl.ds(base, _ROWS_PER_W)], idx_v)
    bufs = (rows_a, rows_b)
    sems = (sem_a, sem_b)
    cps = []
    for c in range(_NCH):
        cps.append(pltpu.async_copy(
            wr_hbm.at[idx_v.at[pl.ds(c * _CH, _CH)]], bufs[c % 2], sems[c % 2]))
        if c > 0:
            cps[c - 1].wait()
            pltpu.sync_copy(bufs[(c - 1) % 2],
                            g_out.at[pl.ds(base + (c - 1) * _CH, _CH)])
    cps[_NCH - 1].wait()
    pltpu.sync_copy(bufs[(_NCH - 1) % 2],
                    g_out.at[pl.ds(base + (_NCH - 1) * _CH, _CH)])
    zbase = wid * _ZB
    pltpu.sync_copy(qrid_hbm.at[pl.ds(zbase, _ZB)], zidx_v)
    pltpu.async_copy(zq_hbm.at[zidx_v], zrows_v, sem_a).wait()
    pltpu.sync_copy(zrows_v, zq_out.at[pl.ds(zbase, _ZB)])


@jax.jit
def _sc_gather(w_r_table, rid_flat, zq_table, qrid):
    mesh = plsc.VectorSubcoreMesh(core_axis_name="c", subcore_axis_name="s")
    return pl.kernel(
        _sc_gather_body,
        mesh=mesh,
        out_type=[
            jax.ShapeDtypeStruct((_B * _M, _D), jnp.float32),
            jax.ShapeDtypeStruct((_B, _D), jnp.float32),
        ],
        scratch_types=[
            pltpu.VMEM((_ROWS_PER_W,), jnp.int32),
            pltpu.VMEM((_CH, _D), jnp.float32),
            pltpu.VMEM((_CH, _D), jnp.float32),
            pltpu.VMEM((_ZB,), jnp.int32),
            pltpu.VMEM((_ZB, _D), jnp.float32),
            pltpu.SemaphoreType.DMA,
            pltpu.SemaphoreType.DMA,
        ],
    )(w_r_table, rid_flat, zq_table, qrid)


def _tc_body(e_ref, g_ref, zq_ref, rw_ref, wattn_ref, battn_ref, wua_ref,
             out_ref):
    bb, m, d = e_ref.shape
    e = e_ref[...].reshape(bb * m, d)
    g = g_ref[...].reshape(bb * m, d)
    s = jnp.sum(g * g, axis=1, keepdims=True)
    de = jnp.sum(e * g, axis=1, keepdims=True)
    c = de / jnp.maximum(s, 1e-24)
    etr = e - c * g
    w1 = wattn_ref[:, :d].astype(jnp.bfloat16)
    w2 = wattn_ref[:, d:].astype(jnp.bfloat16)
    zqw = lax.dot_general(zq_ref[...].astype(jnp.bfloat16), w1,
                          (((1,), (1,)), ((), ())),
                          preferred_element_type=jnp.float32)
    zqw = zqw + battn_ref[...]
    wa = lax.dot_general(etr.astype(jnp.bfloat16), w2,
                         (((1,), (1,)), ((), ())),
                         preferred_element_type=jnp.float32)
    t = jnp.tanh(wa.reshape(bb, m, 2 * d) + zqw[:, None, :])
    ua8 = jnp.broadcast_to(wua_ref[...].astype(jnp.bfloat16), (8, 2 * d))
    logits = lax.dot_general(t.reshape(bb * m, 2 * d).astype(jnp.bfloat16),
                             ua8, (((1,), (1,)), ((), ())),
                             preferred_element_type=jnp.float32)
    ex = jnp.exp(logits[:, :1])
    z = jnp.sum(ex.reshape(bb, m, 1), axis=1)
    inv_z = (1.0 / z).reshape(bb, 1, 1)
    attn = ex * jnp.broadcast_to(inv_z, (bb, m, 1)).reshape(bb * m, 1)
    attn = attn + rw_ref[...]
    out_ref[...] = jnp.sum((attn * etr).reshape(bb, m, d), axis=1)


@jax.jit
def _tc_attention(e, g, zq_rows, rw, W_attn, b_attn2d, W_ua):
    grid = (_B // _BB,)
    return pl.pallas_call(
        _tc_body,
        grid=grid,
        in_specs=[
            pl.BlockSpec((_BB, _M, _D), lambda i: (i, 0, 0)),
            pl.BlockSpec((_BB, _M, _D), lambda i: (i, 0, 0)),
            pl.BlockSpec((_BB, _D), lambda i: (i, 0)),
            pl.BlockSpec((_BB * _M, 1), lambda i: (i, 0)),
            pl.BlockSpec((2 * _D, 2 * _D), lambda i: (0, 0)),
            pl.BlockSpec((1, 2 * _D), lambda i: (0, 0)),
            pl.BlockSpec((1, 2 * _D), lambda i: (0, 0)),
        ],
        out_specs=pl.BlockSpec((_BB, _D), lambda i: (i, 0)),
        out_shape=jax.ShapeDtypeStruct((_B, _D), jnp.float32),
    )(e, g, zq_rows, rw, W_attn, b_attn2d, W_ua)


def kernel(batch_nei_rid, batch_nei_e_emb, batch_nei_rw, batch_q_rid,
           w_r_table, zq_table, W_attn, b_attn, W_ua, b_ua):
    del b_ua
    rid_flat = batch_nei_rid.reshape(_B * _M).astype(jnp.int32)
    qrid = batch_q_rid.astype(jnp.int32)
    g_flat, zq_rows = _sc_gather(w_r_table, rid_flat, zq_table, qrid)
    g = g_flat.reshape(_B, _M, _D)
    rw_col = batch_nei_rw.reshape(_B * _M, 1)
    return _tc_attention(batch_nei_e_emb, g, zq_rows, rw_col,
                         W_attn, b_attn.reshape(1, 2 * _D), W_ua)
